# Initial kernel scaffold; baseline (speedup 1.0000x reference)
import jax
import jax.numpy as jnp
from jax.experimental import pallas as pl


def kernel(weight, token_ids):
    raise NotImplementedError("write your pallas kernel here")


# speedup vs baseline: 3.4370x; 3.4370x over previous
import functools

import jax
import jax.numpy as jnp
from jax import lax
from jax.experimental import pallas as pl
from jax.experimental.pallas import tpu as pltpu
from jax.experimental.pallas import tpu_sc as plsc

D = 128
B_TOK = 16384
S = 20
B = B_TOK * S
NC = 2
NS = 16
NW = NC * NS
BPW = B // NW
C = 128
STEPS = BPW // C

_mesh = plsc.VectorSubcoreMesh(core_axis_name="c", subcore_axis_name="s")


@functools.partial(
    pl.kernel,
    mesh=_mesh,
    out_type=jax.ShapeDtypeStruct((B, D), jnp.float32),
    scratch_types=[
        pltpu.VMEM((STEPS, C), jnp.int32),
        pltpu.VMEM((2, C, D), jnp.float32),
        pltpu.SemaphoreType.DMA,
        pltpu.SemaphoreType.DMA,
    ],
)
def _gather_rows(table_hbm, idx_hbm, out_hbm, idx_v, rows_v, gsem, osem):
    cid = lax.axis_index("c")
    sid = lax.axis_index("s")
    wid = sid * NC + cid
    base = wid * BPW

    pltpu.sync_copy(idx_hbm.at[wid], idx_v)

    pltpu.async_copy(table_hbm.at[idx_v.at[0]], rows_v.at[0], gsem)

    def body(g, buf):
        nxt = 1 - buf

        @pl.when(g + 1 < STEPS)
        def _():
            pltpu.async_copy(table_hbm.at[idx_v.at[g + 1]], rows_v.at[nxt], gsem)

        pltpu.make_async_copy(table_hbm.at[idx_v.at[g]], rows_v.at[buf], gsem).wait()
        pltpu.sync_copy(rows_v.at[buf], out_hbm.at[pl.ds(base + g * C, C)])
        return nxt

    lax.fori_loop(0, STEPS, body, 0)


def kernel(weight, token_ids):
    idx = token_ids.reshape(-1).astype(jnp.int32).reshape(NW, STEPS, C)
    out = _gather_rows(weight, idx)
    return out.reshape(B_TOK, S, D)
